# baseline (speedup 1.0000x reference)
import jax
import jax.numpy as jnp
from jax.experimental import pallas as pl
from jax.experimental.pallas import tpu as pltpu


def _make_body(B, N, D, CBM, NBUF):
    NCH = B // CBM

    def body(patch_hbm, pos_hbm, out_hbm, posb, inb, outb,
             possem, insem, outsem):
        def in_cp(c):
            j = c % NBUF
            return pltpu.make_async_copy(
                patch_hbm.at[pl.ds(c * CBM, CBM)], inb.at[j], insem.at[j])

        def out_cp(c):
            j = c % NBUF
            return pltpu.make_async_copy(
                outb.at[j], out_hbm.at[pl.ds(c * CBM, CBM)], outsem.at[j])

        pltpu.make_async_copy(pos_hbm, posb, possem).start()
        for c in range(NBUF):
            in_cp(c).start()
        pltpu.make_async_copy(pos_hbm, posb, possem).wait()

        for c in range(NCH):
            j = c % NBUF
            in_cp(c).wait()
            if c >= NBUF:
                out_cp(c - NBUF).wait()
            outb[j] = inb[j] + posb[...]
            if c + NBUF < NCH:
                in_cp(c + NBUF).start()
            out_cp(c).start()

        for c in range(NCH - NBUF, NCH):
            out_cp(c).wait()

    return body


def kernel(patch, pos_table):
    B, N, D = patch.shape
    CBM = 1
    NBUF = 8
    return pl.pallas_call(
        _make_body(B, N, D, CBM, NBUF),
        in_specs=[
            pl.BlockSpec(memory_space=pl.ANY),
            pl.BlockSpec(memory_space=pl.ANY),
        ],
        out_specs=pl.BlockSpec(memory_space=pl.ANY),
        out_shape=jax.ShapeDtypeStruct((B, N, D), patch.dtype),
        scratch_shapes=[
            pltpu.VMEM((N, D), patch.dtype),
            pltpu.VMEM((NBUF, CBM, N, D), patch.dtype),
            pltpu.VMEM((NBUF, CBM, N, D), patch.dtype),
            pltpu.SemaphoreType.DMA,
            pltpu.SemaphoreType.DMA((NBUF,)),
            pltpu.SemaphoreType.DMA((NBUF,)),
        ],
    )(patch, pos_table)


# speedup vs baseline: 1.0096x; 1.0096x over previous
import jax
import jax.numpy as jnp
from jax.experimental import pallas as pl
from jax.experimental.pallas import tpu as pltpu


def _add_kernel(patch_ref, pos_ref, out_ref):
    out_ref[...] = patch_ref[...] + pos_ref[...]


def kernel(patch, pos_table):
    B, N, D = patch.shape
    CB = 8
    return pl.pallas_call(
        _add_kernel,
        grid=(B // CB,),
        in_specs=[
            pl.BlockSpec((CB, N, D), lambda b: (b, 0, 0)),
            pl.BlockSpec((N, D), lambda b: (0, 0)),
        ],
        out_specs=pl.BlockSpec((CB, N, D), lambda b: (b, 0, 0)),
        out_shape=jax.ShapeDtypeStruct((B, N, D), patch.dtype),
        compiler_params=pltpu.CompilerParams(
            dimension_semantics=("arbitrary",),
            vmem_limit_bytes=128 * 1024 * 1024,
        ),
    )(patch, pos_table)
